# baseline (speedup 1.0000x reference)
import functools

import jax
import jax.numpy as jnp
from jax import lax
from jax.experimental import pallas as pl
from jax.experimental.pallas import tpu as pltpu
from jax.experimental.pallas import tpu_sc as plsc

NC = 2
NS = 16
NW = NC * NS

IR = 128
CR = 4


def _build(n_rows, d_w, d_f, n_f_rows, vocab):
  rows_per_w = n_rows // NW
  n_chunks = rows_per_w // CR
  d_out = d_w + d_f
  n_tok = n_rows * IR
  mesh = plsc.VectorSubcoreMesh(
      core_axis_name="c", subcore_axis_name="s",
      num_cores=NC, num_subcores=NS)

  @functools.partial(
      pl.kernel,
      out_type=jax.ShapeDtypeStruct((n_tok, d_out), jnp.float32),
      mesh=mesh,
      scratch_types=[
          pltpu.VMEM((CR, IR), jnp.int32),
          pltpu.VMEM((CR, IR), jnp.int32),
          pltpu.VMEM((CR * IR, d_w), jnp.float32),
          pltpu.VMEM((CR * IR, d_f), jnp.float32),
          pltpu.SemaphoreType.DMA,
          pltpu.SemaphoreType.DMA,
      ],
      compiler_params=pltpu.CompilerParams(use_tc_tiling_on_sc=False),
  )
  def body(x_hbm, y_hbm, wv_hbm, ft_hbm, out_hbm,
           xi_v, yi_v, wbuf, fbuf, sem_w, sem_f):
    wid = lax.axis_index("s") * NC + lax.axis_index("c")
    row0 = wid * rows_per_w

    def chunk(g, carry):
      r = row0 + g * CR
      pltpu.sync_copy(x_hbm.at[pl.ds(r, CR)], xi_v)
      pltpu.sync_copy(y_hbm.at[pl.ds(r, CR)], yi_v)
      handles = []
      for j in range(CR):
        handles.append(pltpu.async_copy(
            wv_hbm.at[xi_v.at[j]], wbuf.at[pl.ds(j * IR, IR)], sem_w))
        handles.append(pltpu.async_copy(
            ft_hbm.at[yi_v.at[j]], fbuf.at[pl.ds(j * IR, IR)], sem_f))
      for h in handles:
        h.wait()
      t0 = r * IR
      pltpu.sync_copy(wbuf, out_hbm.at[pl.ds(t0, CR * IR), pl.ds(0, d_w)])
      pltpu.sync_copy(fbuf, out_hbm.at[pl.ds(t0, CR * IR), pl.ds(d_w, d_f)])
      return carry

    lax.fori_loop(0, n_chunks, chunk, 0)

  return body


def kernel(x, y, word_vectors, f_table):
  b, h = x.shape
  n_tok = b * h
  d_w = word_vectors.shape[1]
  d_f = f_table.shape[1]
  n_rows = n_tok // IR
  x2 = x.reshape(n_rows, IR).astype(jnp.int32)
  y2 = y.reshape(n_rows, IR).astype(jnp.int32)
  body = _build(n_rows, d_w, d_f, f_table.shape[0], word_vectors.shape[0])
  out = body(x2, y2, word_vectors, f_table)
  return out.reshape(b, h, d_w + d_f)
